# baseline (speedup 1.0000x reference)
import functools

import jax
import jax.numpy as jnp
from jax.experimental import pallas as pl
from jax.experimental.pallas import tpu as pltpu

NUM_EXPERTS = 8
TOP_K = 2
HIDDEN = 1024
FFN = 4096
CAP = 1024
FB = 8
FBS = FFN // FB


def _mlp_body(g_ref, w1_ref, w2_ref, o_ref):
    fb = pl.program_id(1)
    g = g_ref[...].astype(jnp.bfloat16)
    w1 = w1_ref[0].astype(jnp.bfloat16)
    h = jnp.dot(g, w1, preferred_element_type=jnp.float32)
    h = jax.nn.gelu(h)
    w2 = w2_ref[0].astype(jnp.bfloat16)
    contrib = jnp.dot(h.astype(jnp.bfloat16), w2,
                      preferred_element_type=jnp.float32)

    @pl.when(fb == 0)
    def _():
        o_ref[...] = contrib

    @pl.when(fb != 0)
    def _():
        o_ref[...] += contrib


def _expert_mlp(gathered, w1, w2):
    return pl.pallas_call(
        _mlp_body,
        grid=(NUM_EXPERTS, FB),
        in_specs=[
            pl.BlockSpec((CAP, HIDDEN), lambda e, fb: (e, 0)),
            pl.BlockSpec((1, HIDDEN, FBS), lambda e, fb: (e, 0, fb)),
            pl.BlockSpec((1, FBS, HIDDEN), lambda e, fb: (e, fb, 0)),
        ],
        out_specs=pl.BlockSpec((CAP, HIDDEN), lambda e, fb: (e, 0)),
        out_shape=jax.ShapeDtypeStruct((NUM_EXPERTS * CAP, HIDDEN),
                                       jnp.float32),
    )(gathered, w1, w2)


def kernel(x, scores, logits, expert_weights, top_experts, w1, w2):
    bs, sl, hs = x.shape
    tokens = bs * sl
    xf = x.reshape(tokens, hs)
    te = top_experts.reshape(-1).astype(jnp.int32)
    ew = expert_weights.reshape(-1)

    order = jnp.argsort(te, stable=True)
    tokens_per_expert = jnp.bincount(te, length=NUM_EXPERTS)
    starts = jnp.cumsum(tokens_per_expert) - tokens_per_expert
    rank = jnp.zeros((tokens * TOP_K,), jnp.int32).at[order].set(
        jnp.arange(tokens * TOP_K, dtype=jnp.int32) - starts[te[order]])
    kept = rank < CAP
    slot = te * CAP + rank
    slot_t = jnp.where(kept, slot, NUM_EXPERTS * CAP)
    src_tok = jnp.zeros((NUM_EXPERTS * CAP + 1,), jnp.int32).at[slot_t].set(
        jnp.arange(tokens * TOP_K, dtype=jnp.int32) // TOP_K)[:NUM_EXPERTS * CAP]
    inv_slot = jnp.where(kept, slot, 0)
    wq = jnp.where(kept, ew, 0.0)

    gathered = jnp.take(xf, src_tok, axis=0)
    out_e = _expert_mlp(gathered, w1, w2)

    rows = jnp.take(out_e, inv_slot.reshape(tokens, TOP_K), axis=0)
    out = jnp.sum(rows * wq.reshape(tokens, TOP_K)[..., None], axis=1)
    return out.reshape(bs, sl, hs)


# speedup vs baseline: 1.0184x; 1.0184x over previous
import functools

import jax
import jax.numpy as jnp
from jax.experimental import pallas as pl
from jax.experimental.pallas import tpu as pltpu

NUM_EXPERTS = 8
TOP_K = 2
HIDDEN = 1024
FFN = 4096
CAP = 1024
FB = 8
FBS = FFN // FB


def _mlp_body(g_ref, w1_ref, w2_ref, o_ref):
    fb = pl.program_id(1)
    h = jnp.dot(g_ref[...], w1_ref[0], preferred_element_type=jnp.float32,
                precision=jax.lax.Precision.DEFAULT)
    h = jax.nn.gelu(h)
    contrib = jnp.dot(h, w2_ref[0], preferred_element_type=jnp.float32,
                      precision=jax.lax.Precision.DEFAULT)

    @pl.when(fb == 0)
    def _():
        o_ref[...] = contrib

    @pl.when(fb != 0)
    def _():
        o_ref[...] += contrib


def _expert_mlp(gathered, w1, w2):
    return pl.pallas_call(
        _mlp_body,
        grid=(NUM_EXPERTS, FB),
        in_specs=[
            pl.BlockSpec((CAP, HIDDEN), lambda e, fb: (e, 0)),
            pl.BlockSpec((1, HIDDEN, FBS), lambda e, fb: (e, 0, fb)),
            pl.BlockSpec((1, FBS, HIDDEN), lambda e, fb: (e, fb, 0)),
        ],
        out_specs=pl.BlockSpec((CAP, HIDDEN), lambda e, fb: (e, 0)),
        out_shape=jax.ShapeDtypeStruct((NUM_EXPERTS * CAP, HIDDEN),
                                       jnp.float32),
    )(gathered, w1, w2)


def kernel(x, scores, logits, expert_weights, top_experts, w1, w2):
    bs, sl, hs = x.shape
    tokens = bs * sl
    xf = x.reshape(tokens, hs)
    te = top_experts.reshape(-1).astype(jnp.int32)
    ew = expert_weights.reshape(-1)

    order = jnp.argsort(te, stable=True)
    tokens_per_expert = jnp.bincount(te, length=NUM_EXPERTS)
    starts = jnp.cumsum(tokens_per_expert) - tokens_per_expert
    rank = jnp.zeros((tokens * TOP_K,), jnp.int32).at[order].set(
        jnp.arange(tokens * TOP_K, dtype=jnp.int32) - starts[te[order]])
    kept = rank < CAP
    slot = te * CAP + rank
    slot_t = jnp.where(kept, slot, NUM_EXPERTS * CAP)
    src_tok = jnp.zeros((NUM_EXPERTS * CAP + 1,), jnp.int32).at[slot_t].set(
        jnp.arange(tokens * TOP_K, dtype=jnp.int32) // TOP_K)[:NUM_EXPERTS * CAP]
    inv_slot = jnp.where(kept, slot, 0)
    wq = jnp.where(kept, ew, 0.0)

    gathered = jnp.take(xf, src_tok, axis=0)
    out_e = _expert_mlp(gathered, w1, w2)

    rows = jnp.take(out_e, inv_slot.reshape(tokens, TOP_K), axis=0)
    out = jnp.sum(rows * wq.reshape(tokens, TOP_K)[..., None], axis=1)
    return out.reshape(bs, sl, hs)


# speedup vs baseline: 1.5690x; 1.5407x over previous
import functools

import jax
import jax.numpy as jnp
from jax import lax
from jax.experimental import pallas as pl
from jax.experimental.pallas import tpu as pltpu
from jax.experimental.pallas import tpu_sc as plsc

NUM_EXPERTS = 8
TOP_K = 2
HIDDEN = 1024
FFN = 4096
CAP = 1024
TOKENS = 4096
A = TOKENS * TOP_K
SLOTS = NUM_EXPERTS * CAP
TRASH = SLOTS
FB = 8
FBS = FFN // FB

_ROUTE_W = 16
_RCHUNK = A // _ROUTE_W
_NW = 32
_GCHUNK = SLOTS // _NW
_GSUB = 64
_CCHUNK = A // _NW
_CSUB = 64


def _route_body(te_hbm, ew_hbm, srctok_hbm, inv_hbm, wq_hbm,
                te_v, ew_v, inv_v, wq_v, cnt_v, hist_v, sidx, sval,
                sh_hist, sh_src):
    c = lax.axis_index("c")
    s = lax.axis_index("s")

    @pl.when(c == 0)
    def _():
        w = s
        base = w * _RCHUNK
        pltpu.sync_copy(te_hbm.at[pl.ds(base, _RCHUNK)], te_v)
        pltpu.sync_copy(ew_hbm.at[pl.ds(base, _RCHUNK)], ew_v)
        ids = lax.iota(jnp.int32, 16)

        cnt = jnp.zeros((16,), jnp.int32)
        for j in range(_RCHUNK // 16):
            v = te_v[pl.ds(j * 16, 16)]
            for e in range(NUM_EXPERTS):
                n = jnp.sum((v == e).astype(jnp.int32))
                cnt = jnp.where(ids == e, cnt + n, cnt)
        cnt_v[...] = cnt
        pltpu.sync_copy(cnt_v, sh_hist.at[w])
        plsc.subcore_barrier()

        pltpu.sync_copy(sh_hist, hist_v)
        wb = jnp.zeros((16,), jnp.int32) + w
        acc = jnp.zeros((16,), jnp.int32)
        for w2 in range(_ROUTE_W):
            row = hist_v[w2]
            acc = acc + jnp.where(jnp.full((16,), w2, jnp.int32) < wb, row, 0)

        cnt = acc
        for j in range(_RCHUNK // 16):
            v = te_v[pl.ds(j * 16, 16)]
            ewv = ew_v[pl.ds(j * 16, 16)]
            cnt_v[...] = cnt
            basev = plsc.load_gather(cnt_v, [v])
            prefix = jnp.zeros((16,), jnp.int32)
            for e in range(NUM_EXPERTS):
                m = v == e
                mi = m.astype(jnp.int32)
                pr = jnp.cumsum(mi)
                prefix = jnp.where(m, pr, prefix)
                n = jnp.sum(mi)
                cnt = jnp.where(ids == e, cnt + n, cnt)
            rank = basev + prefix - 1
            kept = rank < CAP
            slot = v * CAP + rank
            inv_v[pl.ds(j * 16, 16)] = jnp.where(kept, slot, 0)
            wq_v[pl.ds(j * 16, 16)] = jnp.where(kept, ewv, 0.0)
            toks = (base + j * 16 + ids) // TOP_K
            sidx[j // 8, pl.ds((j % 8) * 16, 16)] = jnp.where(kept, slot, TRASH)
            sval[j // 8, pl.ds((j % 8) * 16, 16)] = toks

        pltpu.sync_copy(inv_v, inv_hbm.at[pl.ds(base, _RCHUNK)])
        pltpu.sync_copy(wq_v, wq_hbm.at[pl.ds(base, _RCHUNK)])
        for r in range(4):
            pltpu.sync_copy(sval.at[r], sh_src.at[sidx.at[r]])
        plsc.subcore_barrier()
        pltpu.sync_copy(sh_src.at[pl.ds(base, _RCHUNK)],
                        srctok_hbm.at[pl.ds(base, _RCHUNK)])


def _route(te, ew):
    mesh = plsc.VectorSubcoreMesh(core_axis_name="c", subcore_axis_name="s")
    return pl.kernel(
        _route_body,
        out_type=(
            jax.ShapeDtypeStruct((SLOTS + 128,), jnp.int32),
            jax.ShapeDtypeStruct((A,), jnp.int32),
            jax.ShapeDtypeStruct((A,), jnp.float32),
        ),
        mesh=mesh,
        compiler_params=pltpu.CompilerParams(needs_layout_passes=False),
        scratch_types=[
            pltpu.VMEM((_RCHUNK,), jnp.int32),
            pltpu.VMEM((_RCHUNK,), jnp.float32),
            pltpu.VMEM((_RCHUNK,), jnp.int32),
            pltpu.VMEM((_RCHUNK,), jnp.float32),
            pltpu.VMEM((16,), jnp.int32),
            pltpu.VMEM((16, 16), jnp.int32),
            pltpu.VMEM((4, 128), jnp.int32),
            pltpu.VMEM((4, 128), jnp.int32),
            pltpu.VMEM_SHARED((16, 16), jnp.int32),
            pltpu.VMEM_SHARED((SLOTS + 128,), jnp.int32),
        ],
    )(te, ew)


def _gather_body(src_hbm, xf_hbm, g_hbm, idx_v, idx2, rows, sem):
    c = lax.axis_index("c")
    s = lax.axis_index("s")
    wid = c * 16 + s
    base = wid * _GCHUNK
    pltpu.sync_copy(src_hbm.at[pl.ds(base, _GCHUNK)], idx_v)
    for j in range(_GCHUNK // 16):
        x = idx_v[pl.ds(j * 16, 16)]
        idx2[j // 4, pl.ds((j % 4) * 16, 16)] = jnp.clip(x, 0, TOKENS - 1)
    for k in range(_GCHUNK // _GSUB):
        pltpu.async_copy(xf_hbm.at[idx2.at[k]], rows, sem).wait()
        pltpu.sync_copy(rows, g_hbm.at[pl.ds(base + k * _GSUB, _GSUB)])


def _gather(src_tok, xf):
    mesh = plsc.VectorSubcoreMesh(core_axis_name="c", subcore_axis_name="s")
    return pl.kernel(
        _gather_body,
        out_type=jax.ShapeDtypeStruct((SLOTS, HIDDEN), jnp.float32),
        mesh=mesh,
        compiler_params=pltpu.CompilerParams(needs_layout_passes=False),
        scratch_types=[
            pltpu.VMEM((_GCHUNK,), jnp.int32),
            pltpu.VMEM((_GCHUNK // _GSUB, _GSUB), jnp.int32),
            pltpu.VMEM((_GSUB, HIDDEN), jnp.float32),
            pltpu.SemaphoreType.DMA,
        ],
    )(src_tok, xf)


def _mlp_body(g_ref, w1_ref, w2_ref, o_ref):
    fb = pl.program_id(1)
    h = jnp.dot(g_ref[...], w1_ref[0], preferred_element_type=jnp.float32,
                precision=lax.Precision.DEFAULT)
    h = jax.nn.gelu(h)
    contrib = jnp.dot(h, w2_ref[0], preferred_element_type=jnp.float32,
                      precision=lax.Precision.DEFAULT)

    @pl.when(fb == 0)
    def _():
        o_ref[...] = contrib

    @pl.when(fb != 0)
    def _():
        o_ref[...] += contrib


def _expert_mlp(gathered, w1, w2):
    return pl.pallas_call(
        _mlp_body,
        grid=(NUM_EXPERTS, FB),
        in_specs=[
            pl.BlockSpec((CAP, HIDDEN), lambda e, fb: (e, 0)),
            pl.BlockSpec((1, HIDDEN, FBS), lambda e, fb: (e, 0, fb)),
            pl.BlockSpec((1, FBS, HIDDEN), lambda e, fb: (e, fb, 0)),
        ],
        out_specs=pl.BlockSpec((CAP, HIDDEN), lambda e, fb: (e, 0)),
        out_shape=jax.ShapeDtypeStruct((SLOTS, HIDDEN), jnp.float32),
    )(gathered, w1, w2)


def _combine_body(oe_hbm, inv_hbm, wq_hbm, out_hbm,
                  inv_v, wq_v, idx2, rows, out_v, sem):
    c = lax.axis_index("c")
    s = lax.axis_index("s")
    wid = c * 16 + s
    base = wid * _CCHUNK
    tbase = wid * (_CCHUNK // TOP_K)
    pltpu.sync_copy(inv_hbm.at[pl.ds(base, _CCHUNK)], inv_v)
    pltpu.sync_copy(wq_hbm.at[pl.ds(base, _CCHUNK)], wq_v)
    for j in range(_CCHUNK // 16):
        x = inv_v[pl.ds(j * 16, 16)]
        idx2[j // 4, pl.ds((j % 4) * 16, 16)] = x
    for k in range(_CCHUNK // _CSUB):
        pltpu.async_copy(oe_hbm.at[idx2.at[k]], rows, sem).wait()

        def body(r, _):
            p = jnp.full((16,), k * _CSUB, jnp.int32) + 2 * r
            wa = plsc.load_gather(wq_v, [p])
            wb = plsc.load_gather(wq_v, [p + 1])
            for j in range(HIDDEN // 16):
                a = rows[2 * r, pl.ds(j * 16, 16)]
                b = rows[2 * r + 1, pl.ds(j * 16, 16)]
                out_v[r, pl.ds(j * 16, 16)] = a * wa + b * wb
            return 0

        lax.fori_loop(0, _CSUB // TOP_K, body, 0)
        pltpu.sync_copy(
            out_v, out_hbm.at[pl.ds(tbase + k * (_CSUB // TOP_K),
                                    _CSUB // TOP_K)])


def _combine(out_e, inv_slot, wq):
    mesh = plsc.VectorSubcoreMesh(core_axis_name="c", subcore_axis_name="s")
    return pl.kernel(
        _combine_body,
        out_type=jax.ShapeDtypeStruct((TOKENS, HIDDEN), jnp.float32),
        mesh=mesh,
        compiler_params=pltpu.CompilerParams(needs_layout_passes=False),
        scratch_types=[
            pltpu.VMEM((_CCHUNK,), jnp.int32),
            pltpu.VMEM((_CCHUNK,), jnp.float32),
            pltpu.VMEM((_CCHUNK // _CSUB, _CSUB), jnp.int32),
            pltpu.VMEM((_CSUB, HIDDEN), jnp.float32),
            pltpu.VMEM((_CSUB // TOP_K, HIDDEN), jnp.float32),
            pltpu.SemaphoreType.DMA,
        ],
    )(out_e, inv_slot, wq)


def kernel(x, scores, logits, expert_weights, top_experts, w1, w2):
    bs, sl, hs = x.shape
    xf = x.reshape(bs * sl, hs)
    te = top_experts.reshape(-1).astype(jnp.int32)
    ew = expert_weights.reshape(-1)

    src_tok, inv_slot, wq = _route(te, ew)
    gathered = _gather(src_tok, xf)
    out_e = _expert_mlp(gathered, w1, w2)
    out = _combine(out_e, inv_slot, wq)
    return out.reshape(bs, sl, hs)


# speedup vs baseline: 1.7404x; 1.1092x over previous
import functools

import jax
import jax.numpy as jnp
from jax import lax
from jax.experimental import pallas as pl
from jax.experimental.pallas import tpu as pltpu
from jax.experimental.pallas import tpu_sc as plsc

NUM_EXPERTS = 8
TOP_K = 2
HIDDEN = 1024
FFN = 4096
CAP = 1024
TOKENS = 4096
A = TOKENS * TOP_K
SLOTS = NUM_EXPERTS * CAP
TRASH = SLOTS
FB = 4
FBS = FFN // FB

_ROUTE_W = 16
_RCHUNK = A // _ROUTE_W
_NW = 32
_GCHUNK = SLOTS // _NW
_GSUB = 64
_CCHUNK = A // _NW
_CSUB = 64


def _route_body(te_hbm, ew_hbm, srctok_hbm, inv_hbm, wq_hbm,
                te_v, ew_v, inv_v, wq_v, cnt_v, hist_v, sidx, sval,
                sh_hist, sh_src):
    c = lax.axis_index("c")
    s = lax.axis_index("s")

    @pl.when(c == 0)
    def _():
        w = s
        base = w * _RCHUNK
        pltpu.sync_copy(te_hbm.at[pl.ds(base, _RCHUNK)], te_v)
        pltpu.sync_copy(ew_hbm.at[pl.ds(base, _RCHUNK)], ew_v)
        ids = lax.iota(jnp.int32, 16)

        cnt = jnp.zeros((16,), jnp.int32)
        for j in range(_RCHUNK // 16):
            v = te_v[pl.ds(j * 16, 16)]
            for e in range(NUM_EXPERTS):
                n = jnp.sum((v == e).astype(jnp.int32))
                cnt = jnp.where(ids == e, cnt + n, cnt)
        cnt_v[...] = cnt
        pltpu.sync_copy(cnt_v, sh_hist.at[w])
        plsc.subcore_barrier()

        pltpu.sync_copy(sh_hist, hist_v)
        wb = jnp.zeros((16,), jnp.int32) + w
        acc = jnp.zeros((16,), jnp.int32)
        for w2 in range(_ROUTE_W):
            row = hist_v[w2]
            acc = acc + jnp.where(jnp.full((16,), w2, jnp.int32) < wb, row, 0)

        cnt = acc
        for j in range(_RCHUNK // 16):
            v = te_v[pl.ds(j * 16, 16)]
            ewv = ew_v[pl.ds(j * 16, 16)]
            cnt_v[...] = cnt
            basev = plsc.load_gather(cnt_v, [v])
            prefix = jnp.zeros((16,), jnp.int32)
            for e in range(NUM_EXPERTS):
                m = v == e
                mi = m.astype(jnp.int32)
                pr = jnp.cumsum(mi)
                prefix = jnp.where(m, pr, prefix)
                n = jnp.sum(mi)
                cnt = jnp.where(ids == e, cnt + n, cnt)
            rank = basev + prefix - 1
            kept = rank < CAP
            slot = v * CAP + rank
            inv_v[pl.ds(j * 16, 16)] = jnp.where(kept, slot, 0)
            wq_v[pl.ds(j * 16, 16)] = jnp.where(kept, ewv, 0.0)
            toks = (base + j * 16 + ids) // TOP_K
            sidx[j // 8, pl.ds((j % 8) * 16, 16)] = jnp.where(kept, slot, TRASH)
            sval[j // 8, pl.ds((j % 8) * 16, 16)] = toks

        pltpu.sync_copy(inv_v, inv_hbm.at[pl.ds(base, _RCHUNK)])
        pltpu.sync_copy(wq_v, wq_hbm.at[pl.ds(base, _RCHUNK)])
        for r in range(4):
            pltpu.sync_copy(sval.at[r], sh_src.at[sidx.at[r]])
        plsc.subcore_barrier()
        pltpu.sync_copy(sh_src.at[pl.ds(base, _RCHUNK)],
                        srctok_hbm.at[pl.ds(base, _RCHUNK)])


def _route(te, ew):
    mesh = plsc.VectorSubcoreMesh(core_axis_name="c", subcore_axis_name="s")
    return pl.kernel(
        _route_body,
        out_type=(
            jax.ShapeDtypeStruct((SLOTS + 128,), jnp.int32),
            jax.ShapeDtypeStruct((A,), jnp.int32),
            jax.ShapeDtypeStruct((A,), jnp.float32),
        ),
        mesh=mesh,
        compiler_params=pltpu.CompilerParams(needs_layout_passes=False),
        scratch_types=[
            pltpu.VMEM((_RCHUNK,), jnp.int32),
            pltpu.VMEM((_RCHUNK,), jnp.float32),
            pltpu.VMEM((_RCHUNK,), jnp.int32),
            pltpu.VMEM((_RCHUNK,), jnp.float32),
            pltpu.VMEM((16,), jnp.int32),
            pltpu.VMEM((16, 16), jnp.int32),
            pltpu.VMEM((4, 128), jnp.int32),
            pltpu.VMEM((4, 128), jnp.int32),
            pltpu.VMEM_SHARED((16, 16), jnp.int32),
            pltpu.VMEM_SHARED((SLOTS + 128,), jnp.int32),
        ],
    )(te, ew)


def _gather_body(src_hbm, xf_hbm, g_hbm, idx_v, idx2, rows, sem):
    c = lax.axis_index("c")
    s = lax.axis_index("s")
    wid = c * 16 + s
    base = wid * _GCHUNK
    pltpu.sync_copy(src_hbm.at[pl.ds(base, _GCHUNK)], idx_v)
    for j in range(_GCHUNK // 16):
        x = idx_v[pl.ds(j * 16, 16)]
        idx2[j // 4, pl.ds((j % 4) * 16, 16)] = jnp.clip(x, 0, TOKENS - 1)
    for k in range(_GCHUNK // _GSUB):
        pltpu.async_copy(xf_hbm.at[idx2.at[k]], rows, sem).wait()
        pltpu.sync_copy(rows, g_hbm.at[pl.ds(base + k * _GSUB, _GSUB)])


def _gather(src_tok, xf):
    mesh = plsc.VectorSubcoreMesh(core_axis_name="c", subcore_axis_name="s")
    return pl.kernel(
        _gather_body,
        out_type=jax.ShapeDtypeStruct((SLOTS, HIDDEN), jnp.float32),
        mesh=mesh,
        compiler_params=pltpu.CompilerParams(needs_layout_passes=False),
        scratch_types=[
            pltpu.VMEM((_GCHUNK,), jnp.int32),
            pltpu.VMEM((_GCHUNK // _GSUB, _GSUB), jnp.int32),
            pltpu.VMEM((_GSUB, HIDDEN), jnp.float32),
            pltpu.SemaphoreType.DMA,
        ],
    )(src_tok, xf)


def _mlp_body(g_ref, w1_ref, w2_ref, o_ref):
    fb = pl.program_id(1)
    h = jnp.dot(g_ref[...], w1_ref[0], preferred_element_type=jnp.float32,
                precision=lax.Precision.DEFAULT)
    h = jax.nn.gelu(h)
    contrib = jnp.dot(h, w2_ref[0], preferred_element_type=jnp.float32,
                      precision=lax.Precision.DEFAULT)

    @pl.when(fb == 0)
    def _():
        o_ref[...] = contrib

    @pl.when(fb != 0)
    def _():
        o_ref[...] += contrib


def _expert_mlp(gathered, w1, w2):
    return pl.pallas_call(
        _mlp_body,
        grid=(NUM_EXPERTS, FB),
        in_specs=[
            pl.BlockSpec((CAP, HIDDEN), lambda e, fb: (e, 0)),
            pl.BlockSpec((1, HIDDEN, FBS), lambda e, fb: (e, 0, fb)),
            pl.BlockSpec((1, FBS, HIDDEN), lambda e, fb: (e, fb, 0)),
        ],
        out_specs=pl.BlockSpec((CAP, HIDDEN), lambda e, fb: (e, 0)),
        out_shape=jax.ShapeDtypeStruct((SLOTS, HIDDEN), jnp.float32),
    )(gathered, w1, w2)


def _combine_body(oe_hbm, inv_hbm, wq_hbm, out_hbm,
                  inv_v, wq_v, idx2, rows, out_v, sem):
    c = lax.axis_index("c")
    s = lax.axis_index("s")
    wid = c * 16 + s
    base = wid * _CCHUNK
    tbase = wid * (_CCHUNK // TOP_K)
    pltpu.sync_copy(inv_hbm.at[pl.ds(base, _CCHUNK)], inv_v)
    pltpu.sync_copy(wq_hbm.at[pl.ds(base, _CCHUNK)], wq_v)
    for j in range(_CCHUNK // 16):
        x = inv_v[pl.ds(j * 16, 16)]
        idx2[j // 4, pl.ds((j % 4) * 16, 16)] = x
    for k in range(_CCHUNK // _CSUB):
        pltpu.async_copy(oe_hbm.at[idx2.at[k]], rows, sem).wait()

        def body(r, _):
            p = jnp.full((16,), k * _CSUB, jnp.int32) + 2 * r
            wa = plsc.load_gather(wq_v, [p])
            wb = plsc.load_gather(wq_v, [p + 1])
            for j in range(HIDDEN // 16):
                a = rows[2 * r, pl.ds(j * 16, 16)]
                b = rows[2 * r + 1, pl.ds(j * 16, 16)]
                out_v[r, pl.ds(j * 16, 16)] = a * wa + b * wb
            return 0

        lax.fori_loop(0, _CSUB // TOP_K, body, 0)
        pltpu.sync_copy(
            out_v, out_hbm.at[pl.ds(tbase + k * (_CSUB // TOP_K),
                                    _CSUB // TOP_K)])


def _combine(out_e, inv_slot, wq):
    mesh = plsc.VectorSubcoreMesh(core_axis_name="c", subcore_axis_name="s")
    return pl.kernel(
        _combine_body,
        out_type=jax.ShapeDtypeStruct((TOKENS, HIDDEN), jnp.float32),
        mesh=mesh,
        compiler_params=pltpu.CompilerParams(needs_layout_passes=False),
        scratch_types=[
            pltpu.VMEM((_CCHUNK,), jnp.int32),
            pltpu.VMEM((_CCHUNK,), jnp.float32),
            pltpu.VMEM((_CCHUNK // _CSUB, _CSUB), jnp.int32),
            pltpu.VMEM((_CSUB, HIDDEN), jnp.float32),
            pltpu.VMEM((_CSUB // TOP_K, HIDDEN), jnp.float32),
            pltpu.SemaphoreType.DMA,
        ],
    )(out_e, inv_slot, wq)


def kernel(x, scores, logits, expert_weights, top_experts, w1, w2):
    bs, sl, hs = x.shape
    xf = x.reshape(bs * sl, hs)
    te = top_experts.reshape(-1).astype(jnp.int32)
    ew = expert_weights.reshape(-1)

    src_tok, inv_slot, wq = _route(te, ew)
    gathered = _gather(src_tok, xf)
    out_e = _expert_mlp(gathered, w1, w2)
    out = _combine(out_e, inv_slot, wq)
    return out.reshape(bs, sl, hs)


# speedup vs baseline: 1.7930x; 1.0303x over previous
import functools

import jax
import jax.numpy as jnp
from jax import lax
from jax.experimental import pallas as pl
from jax.experimental.pallas import tpu as pltpu
from jax.experimental.pallas import tpu_sc as plsc

NUM_EXPERTS = 8
TOP_K = 2
HIDDEN = 1024
FFN = 4096
CAP = 1024
TOKENS = 4096
A = TOKENS * TOP_K
SLOTS = NUM_EXPERTS * CAP
TRASH = SLOTS
FB = 2
FBS = FFN // FB

_ROUTE_W = 16
_RCHUNK = A // _ROUTE_W
_NW = 32
_GCHUNK = SLOTS // _NW
_GSUB = 64
_CCHUNK = A // _NW
_CSUB = 64


def _route_body(te_hbm, ew_hbm, srctok_hbm, inv_hbm, wq_hbm,
                te_v, ew_v, inv_v, wq_v, cnt_v, hist_v, sidx, sval,
                sh_hist, sh_src):
    c = lax.axis_index("c")
    s = lax.axis_index("s")

    @pl.when(c == 0)
    def _():
        w = s
        base = w * _RCHUNK
        pltpu.sync_copy(te_hbm.at[pl.ds(base, _RCHUNK)], te_v)
        pltpu.sync_copy(ew_hbm.at[pl.ds(base, _RCHUNK)], ew_v)
        ids = lax.iota(jnp.int32, 16)

        cnt = jnp.zeros((16,), jnp.int32)
        for j in range(_RCHUNK // 16):
            v = te_v[pl.ds(j * 16, 16)]
            for e in range(NUM_EXPERTS):
                n = jnp.sum((v == e).astype(jnp.int32))
                cnt = jnp.where(ids == e, cnt + n, cnt)
        cnt_v[...] = cnt
        pltpu.sync_copy(cnt_v, sh_hist.at[w])
        plsc.subcore_barrier()

        pltpu.sync_copy(sh_hist, hist_v)
        wb = jnp.zeros((16,), jnp.int32) + w
        acc = jnp.zeros((16,), jnp.int32)
        for w2 in range(_ROUTE_W):
            row = hist_v[w2]
            acc = acc + jnp.where(jnp.full((16,), w2, jnp.int32) < wb, row, 0)

        cnt = acc
        for j in range(_RCHUNK // 16):
            v = te_v[pl.ds(j * 16, 16)]
            ewv = ew_v[pl.ds(j * 16, 16)]
            cnt_v[...] = cnt
            basev = plsc.load_gather(cnt_v, [v])
            prefix = jnp.zeros((16,), jnp.int32)
            for e in range(NUM_EXPERTS):
                m = v == e
                mi = m.astype(jnp.int32)
                pr = jnp.cumsum(mi)
                prefix = jnp.where(m, pr, prefix)
                n = jnp.sum(mi)
                cnt = jnp.where(ids == e, cnt + n, cnt)
            rank = basev + prefix - 1
            kept = rank < CAP
            slot = v * CAP + rank
            inv_v[pl.ds(j * 16, 16)] = jnp.where(kept, slot, 0)
            wq_v[pl.ds(j * 16, 16)] = jnp.where(kept, ewv, 0.0)
            toks = (base + j * 16 + ids) // TOP_K
            sidx[j // 8, pl.ds((j % 8) * 16, 16)] = jnp.where(kept, slot, TRASH)
            sval[j // 8, pl.ds((j % 8) * 16, 16)] = toks

        pltpu.sync_copy(inv_v, inv_hbm.at[pl.ds(base, _RCHUNK)])
        pltpu.sync_copy(wq_v, wq_hbm.at[pl.ds(base, _RCHUNK)])
        for r in range(4):
            pltpu.sync_copy(sval.at[r], sh_src.at[sidx.at[r]])
        plsc.subcore_barrier()
        pltpu.sync_copy(sh_src.at[pl.ds(base, _RCHUNK)],
                        srctok_hbm.at[pl.ds(base, _RCHUNK)])


def _route(te, ew):
    mesh = plsc.VectorSubcoreMesh(core_axis_name="c", subcore_axis_name="s")
    return pl.kernel(
        _route_body,
        out_type=(
            jax.ShapeDtypeStruct((SLOTS + 128,), jnp.int32),
            jax.ShapeDtypeStruct((A,), jnp.int32),
            jax.ShapeDtypeStruct((A,), jnp.float32),
        ),
        mesh=mesh,
        compiler_params=pltpu.CompilerParams(needs_layout_passes=False),
        scratch_types=[
            pltpu.VMEM((_RCHUNK,), jnp.int32),
            pltpu.VMEM((_RCHUNK,), jnp.float32),
            pltpu.VMEM((_RCHUNK,), jnp.int32),
            pltpu.VMEM((_RCHUNK,), jnp.float32),
            pltpu.VMEM((16,), jnp.int32),
            pltpu.VMEM((16, 16), jnp.int32),
            pltpu.VMEM((4, 128), jnp.int32),
            pltpu.VMEM((4, 128), jnp.int32),
            pltpu.VMEM_SHARED((16, 16), jnp.int32),
            pltpu.VMEM_SHARED((SLOTS + 128,), jnp.int32),
        ],
    )(te, ew)


def _gather_body(src_hbm, xf_hbm, g_hbm, idx_v, idx2, rows, sem):
    c = lax.axis_index("c")
    s = lax.axis_index("s")
    wid = c * 16 + s
    base = wid * _GCHUNK
    pltpu.sync_copy(src_hbm.at[pl.ds(base, _GCHUNK)], idx_v)
    for j in range(_GCHUNK // 16):
        x = idx_v[pl.ds(j * 16, 16)]
        idx2[j // 4, pl.ds((j % 4) * 16, 16)] = jnp.clip(x, 0, TOKENS - 1)
    for k in range(_GCHUNK // _GSUB):
        pltpu.async_copy(xf_hbm.at[idx2.at[k]], rows, sem).wait()
        pltpu.sync_copy(rows, g_hbm.at[pl.ds(base + k * _GSUB, _GSUB)])


def _gather(src_tok, xf):
    mesh = plsc.VectorSubcoreMesh(core_axis_name="c", subcore_axis_name="s")
    return pl.kernel(
        _gather_body,
        out_type=jax.ShapeDtypeStruct((SLOTS, HIDDEN), jnp.float32),
        mesh=mesh,
        compiler_params=pltpu.CompilerParams(needs_layout_passes=False),
        scratch_types=[
            pltpu.VMEM((_GCHUNK,), jnp.int32),
            pltpu.VMEM((_GCHUNK // _GSUB, _GSUB), jnp.int32),
            pltpu.VMEM((_GSUB, HIDDEN), jnp.float32),
            pltpu.SemaphoreType.DMA,
        ],
    )(src_tok, xf)


def _mlp_body(g_ref, w1_ref, w2_ref, o_ref):
    fb = pl.program_id(1)
    h = jnp.dot(g_ref[...], w1_ref[0], preferred_element_type=jnp.float32,
                precision=lax.Precision.DEFAULT)
    h = jax.nn.gelu(h)
    contrib = jnp.dot(h, w2_ref[0], preferred_element_type=jnp.float32,
                      precision=lax.Precision.DEFAULT)

    @pl.when(fb == 0)
    def _():
        o_ref[...] = contrib

    @pl.when(fb != 0)
    def _():
        o_ref[...] += contrib


def _expert_mlp(gathered, w1, w2):
    return pl.pallas_call(
        _mlp_body,
        grid=(NUM_EXPERTS, FB),
        in_specs=[
            pl.BlockSpec((CAP, HIDDEN), lambda e, fb: (e, 0)),
            pl.BlockSpec((1, HIDDEN, FBS), lambda e, fb: (e, 0, fb)),
            pl.BlockSpec((1, FBS, HIDDEN), lambda e, fb: (e, fb, 0)),
        ],
        out_specs=pl.BlockSpec((CAP, HIDDEN), lambda e, fb: (e, 0)),
        out_shape=jax.ShapeDtypeStruct((SLOTS, HIDDEN), jnp.float32),
    )(gathered, w1, w2)


def _combine_body(oe_hbm, inv_hbm, wq_hbm, out_hbm,
                  inv_v, wq_v, idx2, rows, out_v, sem):
    c = lax.axis_index("c")
    s = lax.axis_index("s")
    wid = c * 16 + s
    base = wid * _CCHUNK
    tbase = wid * (_CCHUNK // TOP_K)
    pltpu.sync_copy(inv_hbm.at[pl.ds(base, _CCHUNK)], inv_v)
    pltpu.sync_copy(wq_hbm.at[pl.ds(base, _CCHUNK)], wq_v)
    for j in range(_CCHUNK // 16):
        x = inv_v[pl.ds(j * 16, 16)]
        idx2[j // 4, pl.ds((j % 4) * 16, 16)] = x
    for k in range(_CCHUNK // _CSUB):
        pltpu.async_copy(oe_hbm.at[idx2.at[k]], rows, sem).wait()

        def body(r, _):
            p = jnp.full((16,), k * _CSUB, jnp.int32) + 2 * r
            wa = plsc.load_gather(wq_v, [p])
            wb = plsc.load_gather(wq_v, [p + 1])
            for j in range(HIDDEN // 16):
                a = rows[2 * r, pl.ds(j * 16, 16)]
                b = rows[2 * r + 1, pl.ds(j * 16, 16)]
                out_v[r, pl.ds(j * 16, 16)] = a * wa + b * wb
            return 0

        lax.fori_loop(0, _CSUB // TOP_K, body, 0)
        pltpu.sync_copy(
            out_v, out_hbm.at[pl.ds(tbase + k * (_CSUB // TOP_K),
                                    _CSUB // TOP_K)])


def _combine(out_e, inv_slot, wq):
    mesh = plsc.VectorSubcoreMesh(core_axis_name="c", subcore_axis_name="s")
    return pl.kernel(
        _combine_body,
        out_type=jax.ShapeDtypeStruct((TOKENS, HIDDEN), jnp.float32),
        mesh=mesh,
        compiler_params=pltpu.CompilerParams(needs_layout_passes=False),
        scratch_types=[
            pltpu.VMEM((_CCHUNK,), jnp.int32),
            pltpu.VMEM((_CCHUNK,), jnp.float32),
            pltpu.VMEM((_CCHUNK // _CSUB, _CSUB), jnp.int32),
            pltpu.VMEM((_CSUB, HIDDEN), jnp.float32),
            pltpu.VMEM((_CSUB // TOP_K, HIDDEN), jnp.float32),
            pltpu.SemaphoreType.DMA,
        ],
    )(out_e, inv_slot, wq)


def kernel(x, scores, logits, expert_weights, top_experts, w1, w2):
    bs, sl, hs = x.shape
    xf = x.reshape(bs * sl, hs)
    te = top_experts.reshape(-1).astype(jnp.int32)
    ew = expert_weights.reshape(-1)

    src_tok, inv_slot, wq = _route(te, ew)
    gathered = _gather(src_tok, xf)
    out_e = _expert_mlp(gathered, w1, w2)
    out = _combine(out_e, inv_slot, wq)
    return out.reshape(bs, sl, hs)


# speedup vs baseline: 1.9271x; 1.0748x over previous
import functools

import jax
import jax.numpy as jnp
from jax import lax
from jax.experimental import pallas as pl
from jax.experimental.pallas import tpu as pltpu
from jax.experimental.pallas import tpu_sc as plsc

NUM_EXPERTS = 8
TOP_K = 2
HIDDEN = 1024
FFN = 4096
CAP = 1024
TOKENS = 4096
A = TOKENS * TOP_K
SLOTS = NUM_EXPERTS * CAP
TRASH = SLOTS
FB = 2
FBS = FFN // FB

_ROUTE_W = 16
_RCHUNK = A // _ROUTE_W
_NW = 32
_GCHUNK = SLOTS // _NW
_GSUB = 32
_CCHUNK = A // _NW
_CSUB = 32


def _route_body(te_hbm, ew_hbm, srctok_hbm, inv_hbm, wq_hbm,
                te_v, ew_v, inv_v, wq_v, cnt_v, hist_v, sidx, sval,
                sh_hist, sh_src):
    c = lax.axis_index("c")
    s = lax.axis_index("s")

    @pl.when(c == 0)
    def _():
        w = s
        base = w * _RCHUNK
        pltpu.sync_copy(te_hbm.at[pl.ds(base, _RCHUNK)], te_v)
        pltpu.sync_copy(ew_hbm.at[pl.ds(base, _RCHUNK)], ew_v)
        ids = lax.iota(jnp.int32, 16)

        cnt = jnp.zeros((16,), jnp.int32)
        for j in range(_RCHUNK // 16):
            v = te_v[pl.ds(j * 16, 16)]
            for e in range(NUM_EXPERTS):
                n = jnp.sum((v == e).astype(jnp.int32))
                cnt = jnp.where(ids == e, cnt + n, cnt)
        cnt_v[...] = cnt
        pltpu.sync_copy(cnt_v, sh_hist.at[w])
        plsc.subcore_barrier()

        pltpu.sync_copy(sh_hist, hist_v)
        wb = jnp.zeros((16,), jnp.int32) + w
        acc = jnp.zeros((16,), jnp.int32)
        for w2 in range(_ROUTE_W):
            row = hist_v[w2]
            acc = acc + jnp.where(jnp.full((16,), w2, jnp.int32) < wb, row, 0)

        cnt = acc
        for j in range(_RCHUNK // 16):
            v = te_v[pl.ds(j * 16, 16)]
            ewv = ew_v[pl.ds(j * 16, 16)]
            cnt_v[...] = cnt
            basev = plsc.load_gather(cnt_v, [v])
            prefix = jnp.zeros((16,), jnp.int32)
            for e in range(NUM_EXPERTS):
                m = v == e
                mi = m.astype(jnp.int32)
                pr = jnp.cumsum(mi)
                prefix = jnp.where(m, pr, prefix)
                n = jnp.sum(mi)
                cnt = jnp.where(ids == e, cnt + n, cnt)
            rank = basev + prefix - 1
            kept = rank < CAP
            slot = v * CAP + rank
            inv_v[pl.ds(j * 16, 16)] = jnp.where(kept, slot, 0)
            wq_v[pl.ds(j * 16, 16)] = jnp.where(kept, ewv, 0.0)
            toks = (base + j * 16 + ids) // TOP_K
            sidx[j // 8, pl.ds((j % 8) * 16, 16)] = jnp.where(kept, slot, TRASH)
            sval[j // 8, pl.ds((j % 8) * 16, 16)] = toks

        pltpu.sync_copy(inv_v, inv_hbm.at[pl.ds(base, _RCHUNK)])
        pltpu.sync_copy(wq_v, wq_hbm.at[pl.ds(base, _RCHUNK)])
        for r in range(4):
            pltpu.sync_copy(sval.at[r], sh_src.at[sidx.at[r]])
        plsc.subcore_barrier()
        pltpu.sync_copy(sh_src.at[pl.ds(base, _RCHUNK)],
                        srctok_hbm.at[pl.ds(base, _RCHUNK)])


def _route(te, ew):
    mesh = plsc.VectorSubcoreMesh(core_axis_name="c", subcore_axis_name="s")
    return pl.kernel(
        _route_body,
        out_type=(
            jax.ShapeDtypeStruct((SLOTS + 128,), jnp.int32),
            jax.ShapeDtypeStruct((A,), jnp.int32),
            jax.ShapeDtypeStruct((A,), jnp.float32),
        ),
        mesh=mesh,
        compiler_params=pltpu.CompilerParams(needs_layout_passes=False),
        scratch_types=[
            pltpu.VMEM((_RCHUNK,), jnp.int32),
            pltpu.VMEM((_RCHUNK,), jnp.float32),
            pltpu.VMEM((_RCHUNK,), jnp.int32),
            pltpu.VMEM((_RCHUNK,), jnp.float32),
            pltpu.VMEM((16,), jnp.int32),
            pltpu.VMEM((16, 16), jnp.int32),
            pltpu.VMEM((4, 128), jnp.int32),
            pltpu.VMEM((4, 128), jnp.int32),
            pltpu.VMEM_SHARED((16, 16), jnp.int32),
            pltpu.VMEM_SHARED((SLOTS + 128,), jnp.int32),
        ],
    )(te, ew)


def _gather_body(src_hbm, xf_hbm, g_hbm, idx_v, idx2, rows2,
                 sin0, sin1, sout0, sout1):
    c = lax.axis_index("c")
    s = lax.axis_index("s")
    wid = c * 16 + s
    base = wid * _GCHUNK
    nk = _GCHUNK // _GSUB
    sin = (sin0, sin1)
    sout = (sout0, sout1)
    pltpu.sync_copy(src_hbm.at[pl.ds(base, _GCHUNK)], idx_v)
    for j in range(_GCHUNK // 16):
        x = idx_v[pl.ds(j * 16, 16)]
        idx2[j // (_GSUB // 16), pl.ds((j % (_GSUB // 16)) * 16, 16)] = (
            jnp.clip(x, 0, TOKENS - 1))

    def gin(k):
        return pltpu.async_copy(xf_hbm.at[idx2.at[k]], rows2.at[k % 2],
                                sin[k % 2])

    def gout(k):
        return pltpu.async_copy(
            rows2.at[k % 2],
            g_hbm.at[pl.ds(base + k * _GSUB, _GSUB)], sout[k % 2])

    cin = {0: gin(0)}
    cout = {}
    for k in range(nk):
        cin[k].wait()
        if k + 1 < nk:
            if k >= 1:
                cout[k - 1].wait()
            cin[k + 1] = gin(k + 1)
        cout[k] = gout(k)
    cout[nk - 1].wait()


def _gather(src_tok, xf):
    mesh = plsc.VectorSubcoreMesh(core_axis_name="c", subcore_axis_name="s")
    return pl.kernel(
        _gather_body,
        out_type=jax.ShapeDtypeStruct((SLOTS, HIDDEN), jnp.float32),
        mesh=mesh,
        compiler_params=pltpu.CompilerParams(needs_layout_passes=False),
        scratch_types=[
            pltpu.VMEM((_GCHUNK,), jnp.int32),
            pltpu.VMEM((_GCHUNK // _GSUB, _GSUB), jnp.int32),
            pltpu.VMEM((2, _GSUB, HIDDEN), jnp.float32),
            pltpu.SemaphoreType.DMA,
            pltpu.SemaphoreType.DMA,
            pltpu.SemaphoreType.DMA,
            pltpu.SemaphoreType.DMA,
        ],
    )(src_tok, xf)


def _mlp_body(g_ref, w1_ref, w2_ref, o_ref):
    fb = pl.program_id(1)
    h = jnp.dot(g_ref[...], w1_ref[0], preferred_element_type=jnp.float32,
                precision=lax.Precision.DEFAULT)
    h = jax.nn.gelu(h)
    contrib = jnp.dot(h, w2_ref[0], preferred_element_type=jnp.float32,
                      precision=lax.Precision.DEFAULT)

    @pl.when(fb == 0)
    def _():
        o_ref[...] = contrib

    @pl.when(fb != 0)
    def _():
        o_ref[...] += contrib


def _expert_mlp(gathered, w1, w2):
    return pl.pallas_call(
        _mlp_body,
        grid=(NUM_EXPERTS, FB),
        in_specs=[
            pl.BlockSpec((CAP, HIDDEN), lambda e, fb: (e, 0)),
            pl.BlockSpec((1, HIDDEN, FBS), lambda e, fb: (e, 0, fb)),
            pl.BlockSpec((1, FBS, HIDDEN), lambda e, fb: (e, fb, 0)),
        ],
        out_specs=pl.BlockSpec((CAP, HIDDEN), lambda e, fb: (e, 0)),
        out_shape=jax.ShapeDtypeStruct((SLOTS, HIDDEN), jnp.float32),
    )(gathered, w1, w2)


def _combine_body(oe_hbm, inv_hbm, wq_hbm, out_hbm,
                  inv_v, wq_v, idx2, rows2, out_v,
                  sin0, sin1, sout0, sout1):
    c = lax.axis_index("c")
    s = lax.axis_index("s")
    wid = c * 16 + s
    base = wid * _CCHUNK
    tbase = wid * (_CCHUNK // TOP_K)
    nk = _CCHUNK // _CSUB
    tsub = _CSUB // TOP_K
    sin = (sin0, sin1)
    sout = (sout0, sout1)
    pltpu.sync_copy(inv_hbm.at[pl.ds(base, _CCHUNK)], inv_v)
    pltpu.sync_copy(wq_hbm.at[pl.ds(base, _CCHUNK)], wq_v)
    for j in range(_CCHUNK // 16):
        x = inv_v[pl.ds(j * 16, 16)]
        idx2[j // (_CSUB // 16), pl.ds((j % (_CSUB // 16)) * 16, 16)] = x

    def gin(k):
        return pltpu.async_copy(oe_hbm.at[idx2.at[k]], rows2.at[k % 2],
                                sin[k % 2])

    cin = {0: gin(0)}
    cout = {}
    for k in range(nk):
        cin[k].wait()
        if k + 1 < nk:
            cin[k + 1] = gin(k + 1)
        if k >= 2:
            cout[k - 2].wait()
        rows = rows2.at[k % 2]

        def body(r, _, k=k, rows=rows):
            p = jnp.full((16,), k * _CSUB, jnp.int32) + 2 * r
            wa = plsc.load_gather(wq_v, [p])
            wb = plsc.load_gather(wq_v, [p + 1])
            for j in range(HIDDEN // 16):
                a = rows[2 * r, pl.ds(j * 16, 16)]
                b = rows[2 * r + 1, pl.ds(j * 16, 16)]
                out_v[k % 2, r, pl.ds(j * 16, 16)] = a * wa + b * wb
            return 0

        lax.fori_loop(0, tsub, body, 0)
        cout[k] = pltpu.async_copy(
            out_v.at[k % 2],
            out_hbm.at[pl.ds(tbase + k * tsub, tsub)], sout[k % 2])
    cout[nk - 2].wait()
    cout[nk - 1].wait()


def _combine(out_e, inv_slot, wq):
    mesh = plsc.VectorSubcoreMesh(core_axis_name="c", subcore_axis_name="s")
    return pl.kernel(
        _combine_body,
        out_type=jax.ShapeDtypeStruct((TOKENS, HIDDEN), jnp.float32),
        mesh=mesh,
        compiler_params=pltpu.CompilerParams(needs_layout_passes=False),
        scratch_types=[
            pltpu.VMEM((_CCHUNK,), jnp.int32),
            pltpu.VMEM((_CCHUNK,), jnp.float32),
            pltpu.VMEM((_CCHUNK // _CSUB, _CSUB), jnp.int32),
            pltpu.VMEM((2, _CSUB, HIDDEN), jnp.float32),
            pltpu.VMEM((2, _CSUB // TOP_K, HIDDEN), jnp.float32),
            pltpu.SemaphoreType.DMA,
            pltpu.SemaphoreType.DMA,
            pltpu.SemaphoreType.DMA,
            pltpu.SemaphoreType.DMA,
        ],
    )(out_e, inv_slot, wq)


def kernel(x, scores, logits, expert_weights, top_experts, w1, w2):
    bs, sl, hs = x.shape
    xf = x.reshape(bs * sl, hs)
    te = top_experts.reshape(-1).astype(jnp.int32)
    ew = expert_weights.reshape(-1)

    src_tok, inv_slot, wq = _route(te, ew)
    gathered = _gather(src_tok, xf)
    out_e = _expert_mlp(gathered, w1, w2)
    out = _combine(out_e, inv_slot, wq)
    return out.reshape(bs, sl, hs)
